# baseline (speedup 1.0000x reference)
import functools

import jax
import jax.numpy as jnp
import numpy as np
from jax import lax
from jax.experimental import pallas as pl
from jax.experimental.pallas import tpu as pltpu
from jax.experimental.pallas import tpu_sc as plsc

_P = 32
_FRAC = 0.2
_B, _C, _H, _W = 16, 3, 512, 512
_HB, _WB = _H // _P, _W // _P
_N = _B * _HB * _WB
_R = _B * _C * _H * _W // _P

_NC, _NS = 2, 16
_NW = _NC * _NS
_ROWS_PER_W = _R // _NW
_CHUNK = 128
_NCHUNK = _ROWS_PER_W // _CHUNK
_K = 4
_SROWS = _K * _CHUNK
_NSUPER = _ROWS_PER_W // _SROWS
_NBUF = 4
_LOOK = 2


def _build_tables():
    rng = np.random.default_rng(0)
    idxs = np.arange(_N)
    shuffled_idxs = rng.permutation(idxs)[: int(_N * _FRAC)]
    _shuffle_idxs = rng.permutation(shuffled_idxs)

    perm = np.arange(_N)
    perm[shuffled_idxs] = _shuffle_idxs

    bp, hbp, wbp = perm // 256, (perm // 16) % 16, perm % 16
    b_n = np.arange(_N) // 256
    hb_n = (np.arange(_N) // 16) % 16
    wb_n = np.arange(_N) % 16
    code = lambda w: (w // 4) * 32 + (w % 4)
    delta = ((bp - b_n) * _C * 8192 + (hbp - hb_n) * 512
             + code(wbp) - code(wb_n)).astype(np.int32)

    mask = np.ones(_N, np.float32)
    mask[shuffled_idxs] = 0.0
    list_same = shuffled_idxs[shuffled_idxs == _shuffle_idxs]
    mask[list_same] = 1.0
    mask = mask.reshape(_B, _HB, _WB)[:, None]
    return delta, mask


_DELTA, _MASK = _build_tables()


@functools.partial(
    pl.kernel,
    out_type=jax.ShapeDtypeStruct((_R, _P), jnp.float32),
    mesh=plsc.VectorSubcoreMesh(core_axis_name="c", subcore_axis_name="s"),
    scratch_types=[
        pltpu.VMEM((_N,), jnp.int32),
        pltpu.VMEM((_NBUF, _SROWS), jnp.int32),
        pltpu.VMEM((_NBUF, _SROWS, _P), jnp.float32),
        [pltpu.SemaphoreType.DMA] * _NBUF,
        [pltpu.SemaphoreType.DMA] * _NBUF,
    ],
    compiler_params=pltpu.CompilerParams(
        use_tc_tiling_on_sc=False,
        needs_layout_passes=False,
        disable_bounds_checks=True,
        disable_semaphore_checks=True,
        skip_device_barrier=True,
    ),
)
def _gather_rows(src_hbm, delta_hbm, out_hbm, delta_v, idx_v, rows_v, gsem, ssem):
    wid = lax.axis_index("s") * _NC + lax.axis_index("c")
    base = wid * _ROWS_PER_W
    gbase = wid * _NCHUNK
    pltpu.sync_copy(delta_hbm, delta_v)
    iota = lax.iota(jnp.int32, 16)
    lane4 = iota % 4

    def fire(sf, bf):
        def chunk_body(k, _):
            g = gbase + sf * _K + k
            q = g % 64
            dbase = ((g // 192) * 16 + q // 4) * 16
            t0 = g * _CHUNK
            for v in range(8):
                patt = dbase + lane4 + 4 * (v // 2)
                dval = plsc.load_gather(delta_v, [patt])
                idx_v.at[bf][pl.ds(k * _CHUNK + 16 * v, 16)] = t0 + 16 * v + iota + dval
            pltpu.async_copy(
                src_hbm.at[idx_v.at[bf].at[pl.ds(k * _CHUNK, _CHUNK)]],
                rows_v.at[bf].at[pl.ds(k * _CHUNK, _CHUNK)],
                gsem[bf],
            )
            return ()

        lax.fori_loop(0, _K, chunk_body, ())

    def drain_gather(bf):
        pltpu.make_async_copy(
            src_hbm.at[pl.ds(0, _SROWS)], rows_v.at[bf], gsem[bf]
        ).wait()

    def drain_store(bf):
        pltpu.make_async_copy(
            rows_v.at[bf], src_hbm.at[pl.ds(0, _SROWS)], ssem[bf]
        ).wait()

    for b in range(_LOOK):
        fire(jnp.int32(b), b)

    def body(t, _):
        for b in range(_NBUF):
            s = t * _NBUF + b
            bf = (b + _LOOK) % _NBUF
            sf = s + _LOOK

            @pl.when(jnp.logical_and(sf < _NSUPER, sf >= _NBUF))
            def _():
                drain_store(bf)

            @pl.when(sf < _NSUPER)
            def _():
                fire(sf, bf)

            drain_gather(b)
            pltpu.async_copy(
                rows_v.at[b], out_hbm.at[pl.ds(base + s * _SROWS, _SROWS)], ssem[b]
            )
        return ()

    lax.fori_loop(0, _NSUPER // _NBUF, body, ())

    for b in range(_NBUF):
        pltpu.make_async_copy(
            rows_v.at[b],
            out_hbm.at[pl.ds(base + (_NSUPER - _NBUF + b) * _SROWS, _SROWS)],
            ssem[b],
        ).wait()


def kernel(image):
    rows = (
        image.reshape(_B, _C, _H // 8, 8, _W // 128, 128)
        .transpose(0, 1, 2, 4, 3, 5)
        .reshape(_R, _P)
    )
    out = _gather_rows(rows, jnp.asarray(_DELTA))
    out = (
        out.reshape(_B, _C, _H // 8, _W // 128, 8, 128)
        .transpose(0, 1, 2, 4, 3, 5)
        .reshape(_B, _C, _H, _W)
    )
    return out, jnp.asarray(_MASK)


# speedup vs baseline: 1.0112x; 1.0112x over previous
import functools

import jax
import jax.numpy as jnp
import numpy as np
from jax import lax
from jax.experimental import pallas as pl
from jax.experimental.pallas import tpu as pltpu
from jax.experimental.pallas import tpu_sc as plsc

_P = 32
_FRAC = 0.2
_B, _C, _H, _W = 16, 3, 512, 512
_HB, _WB = _H // _P, _W // _P
_N = _B * _HB * _WB
_R = _B * _C * _H * _W // _P

_NC, _NS = 2, 16
_NW = _NC * _NS
_ROWS_PER_W = _R // _NW
_CHUNK = 128
_NCHUNK = _ROWS_PER_W // _CHUNK
_K = 4
_SROWS = _K * _CHUNK
_NSUPER = _ROWS_PER_W // _SROWS
_NBUF = 4
_LOOK = 2


def _build_tables():
    rng = np.random.default_rng(0)
    idxs = np.arange(_N)
    shuffled_idxs = rng.permutation(idxs)[: int(_N * _FRAC)]
    _shuffle_idxs = rng.permutation(shuffled_idxs)

    perm = np.arange(_N)
    perm[shuffled_idxs] = _shuffle_idxs

    bp, hbp, wbp = perm // 256, (perm // 16) % 16, perm % 16
    b_n = np.arange(_N) // 256
    hb_n = (np.arange(_N) // 16) % 16
    wb_n = np.arange(_N) % 16
    code = lambda w: (w // 4) * 32 + (w % 4)
    delta = ((bp - b_n) * _C * 8192 + (hbp - hb_n) * 512
             + code(wbp) - code(wb_n)).astype(np.int32)

    G = np.arange(_R // 32)
    n0 = ((G // 768) * 16 + (G % 256) // 16) * 16 + 4 * (G % 4)
    flags = ((delta[n0] == 0) & (delta[n0 + 1] == 0)
             & (delta[n0 + 2] == 0) & (delta[n0 + 3] == 0)).astype(np.int32)
    flags = flags.reshape(_NW, -1)

    mask = np.ones(_N, np.float32)
    mask[shuffled_idxs] = 0.0
    list_same = shuffled_idxs[shuffled_idxs == _shuffle_idxs]
    mask[list_same] = 1.0
    mask = mask.reshape(_B, _HB, _WB)[:, None]
    return delta, flags, mask


_DELTA, _FLAGS, _MASK = _build_tables()
_GPW = _ROWS_PER_W // 32
_GPS = _SROWS // 32


@functools.partial(
    pl.kernel,
    out_type=jax.ShapeDtypeStruct((_R, _P), jnp.float32),
    mesh=plsc.VectorSubcoreMesh(core_axis_name="c", subcore_axis_name="s"),
    scratch_types=[
        pltpu.VMEM((_N,), jnp.int32),
        pltpu.VMEM((_NBUF, _SROWS), jnp.int32),
        pltpu.VMEM((_NBUF, _SROWS, _P), jnp.float32),
        [pltpu.SemaphoreType.DMA] * _NBUF,
        [pltpu.SemaphoreType.DMA] * _NBUF,
    ],
    compiler_params=pltpu.CompilerParams(
        use_tc_tiling_on_sc=False,
        needs_layout_passes=False,
        disable_bounds_checks=True,
        disable_semaphore_checks=True,
    ),
)
def _gather_rows(src_hbm, delta_hbm, out_hbm, delta_v, idx_v, rows_v, gsem, ssem):
    wid = lax.axis_index("s") * _NC + lax.axis_index("c")
    base = wid * _ROWS_PER_W
    gbase = wid * _GPW
    pltpu.sync_copy(delta_hbm, delta_v)
    iota = lax.iota(jnp.int32, 16)
    lane4 = iota % 4

    def fire(sf, bf):
        def group_body(m, _):
            G = gbase + sf * _GPS + m
            t0 = G * 32
            n0 = ((G // 768) * 16 + (G % 256) // 16) * 16 + 4 * (G % 4)
            dval = plsc.load_gather(delta_v, [n0 + lane4])
            moved = lax.reduce_max(jnp.abs(dval), (0,))

            @pl.when(moved == 0)
            def _():
                pltpu.async_copy(
                    src_hbm.at[pl.ds(t0, 32)],
                    rows_v.at[bf].at[pl.ds(m * 32, 32)],
                    gsem[bf],
                )

            @pl.when(moved != 0)
            def _():
                idx_v.at[bf][pl.ds(m * 32, 16)] = t0 + iota + dval
                idx_v.at[bf][pl.ds(m * 32 + 16, 16)] = t0 + 16 + iota + dval
                pltpu.async_copy(
                    src_hbm.at[idx_v.at[bf].at[pl.ds(m * 32, 32)]],
                    rows_v.at[bf].at[pl.ds(m * 32, 32)],
                    gsem[bf],
                )

            return ()

        lax.fori_loop(0, _GPS, group_body, ())

    def drain_gather(bf):
        pltpu.make_async_copy(
            src_hbm.at[pl.ds(0, _SROWS)], rows_v.at[bf], gsem[bf]
        ).wait()

    def drain_store(bf):
        pltpu.make_async_copy(
            rows_v.at[bf], src_hbm.at[pl.ds(0, _SROWS)], ssem[bf]
        ).wait()

    for b in range(_LOOK):
        fire(jnp.int32(b), b)

    def body(t, _):
        for b in range(_NBUF):
            s = t * _NBUF + b
            bf = (b + _LOOK) % _NBUF
            sf = s + _LOOK

            @pl.when(jnp.logical_and(sf < _NSUPER, sf >= _NBUF))
            def _():
                drain_store(bf)

            @pl.when(sf < _NSUPER)
            def _():
                fire(sf, bf)

            drain_gather(b)
            pltpu.async_copy(
                rows_v.at[b], out_hbm.at[pl.ds(base + s * _SROWS, _SROWS)], ssem[b]
            )
        return ()

    lax.fori_loop(0, _NSUPER // _NBUF, body, ())

    for b in range(_NBUF):
        pltpu.make_async_copy(
            rows_v.at[b],
            out_hbm.at[pl.ds(base + (_NSUPER - _NBUF + b) * _SROWS, _SROWS)],
            ssem[b],
        ).wait()


def kernel(image):
    rows = (
        image.reshape(_B, _C, _H // 8, 8, _W // 128, 128)
        .transpose(0, 1, 2, 4, 3, 5)
        .reshape(_R, _P)
    )
    out = _gather_rows(rows, jnp.asarray(_DELTA))
    out = (
        out.reshape(_B, _C, _H // 8, _W // 128, 8, 128)
        .transpose(0, 1, 2, 4, 3, 5)
        .reshape(_B, _C, _H, _W)
    )
    return out, jnp.asarray(_MASK)
